# baseline (speedup 1.0000x reference)
import functools

import jax
import jax.numpy as jnp
from jax import lax
from jax.experimental import pallas as pl
from jax.experimental.pallas import tpu as pltpu
from jax.experimental.pallas import tpu_sc as plsc
from jax.experimental import layout as _layout

_B = 4
_L = 8192
_DH = 64
_MSL = 8192
_PI = 128
_CI = 8
_CT = _DH // _CI
_PT = _L // _PI
_SLAB = _MSL * _CI
_PANEL = _L * _CI
_QRT = _PANEL // 4
_NW = 32

_mesh = plsc.VectorSubcoreMesh(core_axis_name="c", subcore_axis_name="s")


@functools.partial(
    pl.kernel,
    mesh=_mesh,
    compiler_params=pltpu.CompilerParams(use_tc_tiling_on_sc=False,
                                         needs_layout_passes=False,
                                         skip_device_barrier=True,
                                         disable_bounds_checks=True),
    out_type=(
        jax.ShapeDtypeStruct((_B, _CT, _PT, _CI, _PI), jnp.float32),
        jax.ShapeDtypeStruct((_B, _CT, _PT, _CI, _PI), jnp.float32),
    ),
    scratch_types=[
        pltpu.VMEM((_SLAB,), jnp.float32),
        pltpu.VMEM((2 * _PT, _PI), jnp.int32),
        pltpu.VMEM((2, 16, _CI, _PI), jnp.float32),
        pltpu.SemaphoreType.DMA((2,)),
    ],
)
def _rope_gather(pos_hbm, cos_hbm, sin_hbm, cos_out, sin_out,
                 slab_v, pos_v, obuf, sem):
    wid = lax.axis_index("s") * 2 + lax.axis_index("c")
    grp = wid // 2
    tbl = grp // _CT
    ct = grp % _CT
    half = wid % 2

    def run(tbl_hbm, out3):
        pltpu.sync_copy(tbl_hbm.at[ct], slab_v)
        pltpu.sync_copy(pos_hbm.at[pl.ds(half * 2 * _PT, 2 * _PT)], pos_v)
        descs = []
        for qi in range(8):
            b2, qrt = qi // 4, qi % 4
            b = half * 2 + b2
            slot = qi % 2
            if qi >= 2:
                descs[qi - 2].wait()

            @plsc.parallel_loop(0, 128, unroll=2)
            def _(j):
                pt_l = b2 * _PT + qrt * 16 + (j >> 3)
                idx = pos_v[pt_l, pl.ds((j & 7) * 16, 16)]
                base = ((idx >> 7) << 10) + (idx & 127)
                for ci in range(_CI):
                    v = plsc.load_gather(slab_v, [base + ci * _PI])
                    obuf[slot, j >> 3, ci, pl.ds((j & 7) * 16, 16)] = v

            descs.append(pltpu.async_copy(
                obuf.at[slot],
                out3.at[b, ct, pl.ds(qrt * 16, 16)],
                sem.at[slot]))
        for d in descs[-2:]:
            d.wait()

    @pl.when(tbl == 0)
    def _():
        run(cos_hbm, cos_out)

    @pl.when(tbl == 1)
    def _():
        run(sin_hbm, sin_out)


def _to_slabs(table):
    t4 = table.reshape(_MSL // _PI, _PI, _CT, _CI)
    t4 = _layout.with_layout_constraint(
        t4, _layout.Layout((2, 0, 3, 1)))
    return lax.reshape(t4, (_CT, _SLAB), dimensions=(2, 0, 3, 1))


def _from_panels(o5):
    return lax.reshape(o5, (_B, _L, 1, _DH), dimensions=(0, 2, 4, 1, 3))


def kernel(positions, cos_cached, sin_cached):
    pos2 = positions.reshape(_B * _PT, _PI)
    cos_o, sin_o = _rope_gather(pos2, _to_slabs(cos_cached),
                                _to_slabs(sin_cached))
    return (_from_panels(cos_o), _from_panels(sin_o))


# speedup vs baseline: 1.1318x; 1.1318x over previous
import functools

import jax
import jax.numpy as jnp
from jax import lax
from jax.experimental import pallas as pl
from jax.experimental.pallas import tpu as pltpu
from jax.experimental.pallas import tpu_sc as plsc
from jax.experimental import layout as _layout

_B = 4
_L = 8192
_DH = 64
_MSL = 8192
_PI = 128
_CI = 8
_CT = _DH // _CI
_PT = _L // _PI
_SLAB = _MSL * _CI
_PANEL = _L * _CI
_QRT = _PANEL // 4
_NW = 32

_mesh = plsc.VectorSubcoreMesh(core_axis_name="c", subcore_axis_name="s")


@functools.partial(
    pl.kernel,
    mesh=_mesh,
    compiler_params=pltpu.CompilerParams(use_tc_tiling_on_sc=False,
                                         needs_layout_passes=False),
    out_type=(
        jax.ShapeDtypeStruct((_B, _CT, _PT, _CI, _PI), jnp.float32),
        jax.ShapeDtypeStruct((_B, _CT, _PT, _CI, _PI), jnp.float32),
    ),
    scratch_types=[
        pltpu.VMEM((_SLAB,), jnp.float32),
        pltpu.VMEM((2 * _PT, _PI), jnp.int32),
        pltpu.VMEM((2, 16, _CI, _PI), jnp.float32),
        pltpu.SemaphoreType.DMA((2,)),
    ],
)
def _rope_gather(pos_hbm, cos_hbm, sin_hbm, cos_out, sin_out,
                 slab_v, pos_v, obuf, sem):
    wid = lax.axis_index("s") * 2 + lax.axis_index("c")
    grp = wid // 2
    tbl = grp // _CT
    ct = grp % _CT
    half = wid % 2

    def run(tbl_hbm, out3):
        pltpu.sync_copy(tbl_hbm.at[ct], slab_v)
        pltpu.sync_copy(pos_hbm.at[pl.ds(half * 2 * _PT, 2 * _PT)], pos_v)

        def quarter(qi, carry):
            b2 = qi // 4
            qrt = qi % 4
            b = half * 2 + b2
            slot = qi % 2

            @pl.when(qi >= 2)
            def _():
                pltpu.make_async_copy(
                    obuf.at[slot],
                    out3.at[b, ct, pl.ds(qrt * 16, 16)],
                    sem.at[slot]).wait()

            @plsc.parallel_loop(0, 128, unroll=2)
            def _(j):
                pt_l = b2 * _PT + qrt * 16 + (j >> 3)
                idx = pos_v[pt_l, pl.ds((j & 7) * 16, 16)]
                base = ((idx >> 7) << 10) + (idx & 127)
                for ci in range(_CI):
                    v = plsc.load_gather(slab_v, [base + ci * _PI])
                    obuf[slot, j >> 3, ci, pl.ds((j & 7) * 16, 16)] = v

            pltpu.async_copy(
                obuf.at[slot],
                out3.at[b, ct, pl.ds(qrt * 16, 16)],
                sem.at[slot])
            return carry

        lax.fori_loop(0, 8, quarter, 0)
        for slot in range(2):
            pltpu.make_async_copy(
                obuf.at[slot],
                out3.at[0, 0, pl.ds(0, 16)],
                sem.at[slot]).wait()

    @pl.when(tbl == 0)
    def _():
        run(cos_hbm, cos_out)

    @pl.when(tbl == 1)
    def _():
        run(sin_hbm, sin_out)


def _to_slabs(table):
    t4 = table.reshape(_MSL // _PI, _PI, _CT, _CI)
    t4 = _layout.with_layout_constraint(
        t4, _layout.Layout((2, 0, 3, 1)))
    return lax.reshape(t4, (_CT, _SLAB), dimensions=(2, 0, 3, 1))


def _from_panels(o5):
    return lax.reshape(o5, (_B, _L, 1, _DH), dimensions=(0, 2, 4, 1, 3))


def kernel(positions, cos_cached, sin_cached):
    pos2 = positions.reshape(_B * _PT, _PI)
    cos_o, sin_o = _rope_gather(pos2, _to_slabs(cos_cached),
                                _to_slabs(sin_cached))
    return (_from_panels(cos_o), _from_panels(sin_o))


# speedup vs baseline: 1.1372x; 1.0048x over previous
import functools

import jax
import jax.numpy as jnp
from jax import lax
from jax.experimental import pallas as pl
from jax.experimental.pallas import tpu as pltpu
from jax.experimental.pallas import tpu_sc as plsc
from jax.experimental import layout as _layout

_B = 4
_L = 8192
_DH = 64
_MSL = 8192
_PI = 128
_CI = 8
_CT = _DH // _CI
_PT = _L // _PI
_SLAB = _MSL * _CI
_PANEL = _L * _CI
_QRT = _PANEL // 4
_NW = 32

_mesh = plsc.VectorSubcoreMesh(core_axis_name="c", subcore_axis_name="s")


@functools.partial(
    pl.kernel,
    mesh=_mesh,
    compiler_params=pltpu.CompilerParams(use_tc_tiling_on_sc=False,
                                         needs_layout_passes=False),
    out_type=(
        jax.ShapeDtypeStruct((_B, _CT, _PT, _CI, _PI), jnp.float32),
        jax.ShapeDtypeStruct((_B, _CT, _PT, _CI, _PI), jnp.float32),
    ),
    scratch_types=[
        pltpu.VMEM((_SLAB,), jnp.float32),
        pltpu.VMEM((2 * _PT, _PI), jnp.int32),
        pltpu.VMEM((2, 16, _CI, _PI), jnp.float32),
        pltpu.SemaphoreType.DMA((2,)),
    ],
)
def _rope_gather(pos_hbm, cos_hbm, sin_hbm, cos_out, sin_out,
                 slab_v, pos_v, obuf, sem):
    wid = lax.axis_index("s") * 2 + lax.axis_index("c")
    grp = wid // 2
    tbl = grp // _CT
    ct = grp % _CT
    half = wid % 2

    def run(tbl_hbm, out3):
        c_slab = pltpu.async_copy(tbl_hbm.at[ct], slab_v, sem.at[0])
        c_pos = pltpu.async_copy(
            pos_hbm.at[pl.ds(half * 2 * _PT, 2 * _PT)], pos_v, sem.at[1])
        c_pos.wait()
        c_slab.wait()

        def quarter(qi, carry):
            b2 = qi // 4
            qrt = qi % 4
            b = half * 2 + b2
            slot = qi % 2

            @pl.when(qi >= 2)
            def _():
                pltpu.make_async_copy(
                    obuf.at[slot],
                    out3.at[b, ct, pl.ds(qrt * 16, 16)],
                    sem.at[slot]).wait()

            @plsc.parallel_loop(0, 128, unroll=4)
            def _(j):
                pt_l = b2 * _PT + qrt * 16 + (j >> 3)
                idx = pos_v[pt_l, pl.ds((j & 7) * 16, 16)]
                base = ((idx >> 7) << 10) + (idx & 127)
                for ci in range(_CI):
                    v = plsc.load_gather(slab_v, [base + ci * _PI])
                    obuf[slot, j >> 3, ci, pl.ds((j & 7) * 16, 16)] = v

            pltpu.async_copy(
                obuf.at[slot],
                out3.at[b, ct, pl.ds(qrt * 16, 16)],
                sem.at[slot])
            return carry

        lax.fori_loop(0, 8, quarter, 0)
        for slot in range(2):
            pltpu.make_async_copy(
                obuf.at[slot],
                out3.at[0, 0, pl.ds(0, 16)],
                sem.at[slot]).wait()

    @pl.when(tbl == 0)
    def _():
        run(cos_hbm, cos_out)

    @pl.when(tbl == 1)
    def _():
        run(sin_hbm, sin_out)


def _to_slabs(table):
    t4 = table.reshape(_MSL // _PI, _PI, _CT, _CI)
    t4 = _layout.with_layout_constraint(
        t4, _layout.Layout((2, 0, 3, 1)))
    return lax.reshape(t4, (_CT, _SLAB), dimensions=(2, 0, 3, 1))


def _from_panels(o5):
    return lax.reshape(o5, (_B, _L, 1, _DH), dimensions=(0, 2, 4, 1, 3))


def kernel(positions, cos_cached, sin_cached):
    pos2 = positions.reshape(_B * _PT, _PI)
    cos_o, sin_o = _rope_gather(pos2, _to_slabs(cos_cached),
                                _to_slabs(sin_cached))
    return (_from_panels(cos_o), _from_panels(sin_o))
